# Initial kernel scaffold; baseline (speedup 1.0000x reference)
import jax
import jax.numpy as jnp
from jax.experimental import pallas as pl


def kernel(x, edge_index, edge_weight, W_attn, b_attn, W1, b1, W2, b2, g1, beta1, g2, beta2, Wf1, bf1, Wf2, bf2):
    raise NotImplementedError("write your pallas kernel here")


# speedup vs baseline: 11.2675x; 11.2675x over previous
import functools

import jax
import jax.numpy as jnp
from jax import lax
from jax.experimental import pallas as pl
from jax.experimental.pallas import tpu as pltpu
from jax.experimental.pallas import tpu_sc as plsc

N = 10000
E = 320000
F = 128
K = 128
NW = 32
NWIN = E // K
BASE_WIN = NWIN // NW
EXTRA = NWIN - BASE_WIN * NW

_mesh = plsc.VectorSubcoreMesh(core_axis_name="c", subcore_axis_name="s")


@functools.partial(
    pl.kernel, mesh=_mesh,
    out_type=(jax.ShapeDtypeStruct((E,), jnp.float32),
              jax.ShapeDtypeStruct((2 * N,), jnp.float32)),
    scratch_types=[
        pltpu.VMEM((K,), jnp.int32),
        pltpu.VMEM((K,), jnp.int32),
        pltpu.VMEM((K,), jnp.float32),
        pltpu.VMEM((K,), jnp.float32),
        pltpu.VMEM((K,), jnp.float32),
        pltpu.VMEM((K,), jnp.float32),
        pltpu.VMEM((640,), jnp.float32),
        pltpu.VMEM_SHARED((N,), jnp.float32),
        pltpu.SemaphoreType.DMA,
        pltpu.SemaphoreType.DMA,
    ],
)
def _edge_w_deg(src_hbm, dst_hbm, ew_hbm, a1_hbm, a2_hbm, w_out, deg_out,
                sidx, didx, ewv, a1v, a2v, wv, zbuf, deg_sp, sem1, sem2):
    c = lax.axis_index("c")
    s = lax.axis_index("s")
    wid = s * 2 + c

    zero = jnp.zeros((16,), jnp.float32)
    for q in range(40):
        zbuf[pl.ds(q * 16, 16)] = zero
    base = pl.multiple_of(s * 624, 8)
    pltpu.sync_copy(zbuf.at[pl.ds(0, 624)], deg_sp.at[pl.ds(base, 624)])

    @pl.when(s == 0)
    def _():
        pltpu.sync_copy(zbuf.at[pl.ds(0, 16)], deg_sp.at[pl.ds(9984, 16)])

    plsc.subcore_barrier()

    nwin = BASE_WIN + jnp.where(wid < EXTRA, 1, 0)

    def body(j, carry):
        win = wid + NW * j
        off = pl.multiple_of(win * K, K)
        pltpu.sync_copy(src_hbm.at[pl.ds(off, K)], sidx)
        pltpu.sync_copy(dst_hbm.at[pl.ds(off, K)], didx)
        pltpu.sync_copy(ew_hbm.at[pl.ds(off, K)], ewv)
        cp1 = pltpu.async_copy(a1_hbm.at[sidx], a1v, sem1)
        cp2 = pltpu.async_copy(a2_hbm.at[didx], a2v, sem2)
        cp1.wait()
        cp2.wait()
        for q in range(K // 16):
            sl = pl.ds(q * 16, 16)
            z = a1v[sl] + a2v[sl]
            sg = 1.0 / (1.0 + jnp.exp(-z))
            wv[sl] = ewv[sl] * sg
        pltpu.sync_copy(wv, w_out.at[pl.ds(off, K)])
        pltpu.sync_copy(wv, deg_sp.at[didx], add=True)
        return carry

    lax.fori_loop(0, nwin, body, 0)
    plsc.subcore_barrier()

    obase = pl.multiple_of(c * N + s * 624, 8)
    pltpu.sync_copy(deg_sp.at[pl.ds(base, 624)], zbuf.at[pl.ds(0, 624)])
    pltpu.sync_copy(zbuf.at[pl.ds(0, 624)], deg_out.at[pl.ds(obase, 624)])

    @pl.when(s == 0)
    def _():
        otail = pl.multiple_of(c * N + 9984, 8)
        pltpu.sync_copy(deg_sp.at[pl.ds(9984, 16)], zbuf.at[pl.ds(624, 16)])
        pltpu.sync_copy(zbuf.at[pl.ds(624, 16)], deg_out.at[pl.ds(otail, 16)])


@functools.partial(
    pl.kernel, mesh=_mesh,
    out_type=jax.ShapeDtypeStruct((2 * N, F), jnp.float32),
    scratch_types=[
        pltpu.VMEM((K,), jnp.int32),
        pltpu.VMEM((K,), jnp.int32),
        pltpu.VMEM((K,), jnp.float32),
        pltpu.VMEM((K, F), jnp.float32),
        pltpu.VMEM_SHARED((N, F), jnp.float32),
        pltpu.SemaphoreType.DMA,
    ],
)
def _edge_agg(src_hbm, dst_hbm, w_hbm, h_hbm, agg_out,
              sidx, didx, wv, rows, acc_sp, sem):
    c = lax.axis_index("c")
    s = lax.axis_index("s")
    wid = s * 2 + c

    zero = jnp.zeros((16,), jnp.float32)

    def zbody(r, carry):
        for q in range(F // 16):
            rows[r, pl.ds(q * 16, 16)] = zero
        return carry

    lax.fori_loop(0, K, zbody, 0)
    rstart = pl.multiple_of(s * 624, 8)
    for jj in range(5):
        size = 128 if jj < 4 else 112
        pltpu.sync_copy(rows.at[pl.ds(0, size)],
                        acc_sp.at[pl.ds(rstart + jj * 128, size)])

    @pl.when(s == 15)
    def _():
        pltpu.sync_copy(rows.at[pl.ds(0, 16)], acc_sp.at[pl.ds(9984, 16)])

    plsc.subcore_barrier()

    nwin = BASE_WIN + jnp.where(wid < EXTRA, 1, 0)

    def body(j, carry):
        win = wid + NW * j
        off = pl.multiple_of(win * K, K)
        pltpu.sync_copy(src_hbm.at[pl.ds(off, K)], sidx)
        pltpu.sync_copy(dst_hbm.at[pl.ds(off, K)], didx)
        pltpu.sync_copy(w_hbm.at[pl.ds(off, K)], wv)
        pltpu.async_copy(h_hbm.at[sidx], rows, sem).wait()

        def mbody(g, mc):
            gb = pl.multiple_of(g * 16, 16)
            wg = wv[pl.ds(gb, 16)]
            for l in range(16):
                wspl = jnp.broadcast_to(wg[l], (16,))
                r = gb + l
                for q in range(F // 16):
                    sl = pl.ds(q * 16, 16)
                    rows[r, sl] = rows[r, sl] * wspl
            return mc

        lax.fori_loop(0, K // 16, mbody, 0)
        pltpu.sync_copy(rows, acc_sp.at[didx], add=True)
        return carry

    lax.fori_loop(0, nwin, body, 0)
    plsc.subcore_barrier()

    for jj in range(5):
        size = 128 if jj < 4 else 112
        ro = rstart + jj * 128
        oo = pl.multiple_of(c * N + ro, 8)
        pltpu.sync_copy(acc_sp.at[pl.ds(ro, size)], rows.at[pl.ds(0, size)])
        pltpu.sync_copy(rows.at[pl.ds(0, size)], agg_out.at[pl.ds(oo, size)])

    @pl.when(s == 15)
    def _():
        oo = pl.multiple_of(c * N + 9984, 8)
        pltpu.sync_copy(acc_sp.at[pl.ds(9984, 16)], rows.at[pl.ds(0, 16)])
        pltpu.sync_copy(rows.at[pl.ds(0, 16)], agg_out.at[pl.ds(oo, 16)])


def _tc_a_body(x_ref, wcat_ref, b_ref, w1_ref, a12_ref, h1pre_ref):
    x = x_ref[...]
    a12 = lax.dot_general(wcat_ref[...], x, (((1,), (1,)), ((), ())),
                          preferred_element_type=jnp.float32)
    a12_ref[...] = a12 + jnp.concatenate(
        [jnp.zeros((1, 1), jnp.float32), b_ref[...]], axis=0)
    h1pre_ref[...] = jnp.dot(x, w1_ref[...], preferred_element_type=jnp.float32)


def _tc_c_body(degp_ref, h1pre_ref, dinv_ref, h1s_ref):
    deg = degp_ref[:, 0:1] + degp_ref[:, 1:2] + 1.0
    dinv = lax.rsqrt(deg)
    dinv_ref[...] = dinv
    h1s_ref[...] = h1pre_ref[...] * dinv


def _bn_relu(conv, g_ref, beta_ref):
    r = jnp.maximum(conv, 0.0)
    mu = jnp.mean(r, axis=0, keepdims=True)
    var = jnp.mean(r * r, axis=0, keepdims=True) - mu * mu
    return g_ref[...] * (r - mu) * lax.rsqrt(var + 1e-5) + beta_ref[...]


def _tc_e_body(aggp_ref, h1pre_ref, dinv_ref, b1_ref, g1_ref, beta1_ref,
               w2_ref, h_ref, h2pre_ref, h2s_ref):
    dinv = dinv_ref[...]
    conv1 = dinv * (aggp_ref[0:N] + aggp_ref[N:2 * N]) \
        + (dinv * dinv) * h1pre_ref[...] + b1_ref[...]
    h = _bn_relu(conv1, g1_ref, beta1_ref)
    h_ref[...] = h
    h2pre = jnp.dot(h, w2_ref[...], preferred_element_type=jnp.float32)
    h2pre_ref[...] = h2pre
    h2s_ref[...] = h2pre * dinv


def _tc_g_body(aggp_ref, h2pre_ref, dinv_ref, b2_ref, g2_ref, beta2_ref,
               x_ref, h_ref, wf1_ref, bf1_ref, wf2_ref, bf2_ref, out_ref):
    dinv = dinv_ref[...]
    conv2 = dinv * (aggp_ref[0:N] + aggp_ref[N:2 * N]) \
        + (dinv * dinv) * h2pre_ref[...] + b2_ref[...]
    h2 = _bn_relu(conv2, g2_ref, beta2_ref)
    wf1 = wf1_ref[...]
    z = jnp.dot(x_ref[...], wf1[0:F], preferred_element_type=jnp.float32) \
        + jnp.dot(h_ref[...], wf1[F:2 * F], preferred_element_type=jnp.float32) \
        + jnp.dot(h2, wf1[2 * F:3 * F], preferred_element_type=jnp.float32) \
        + bf1_ref[...]
    z = jnp.maximum(z, 0.0)
    out = jnp.dot(z, wf2_ref[...], preferred_element_type=jnp.float32) + bf2_ref[...]
    out_ref[...] = jnp.maximum(out, 0.0)


_tc_a = pl.pallas_call(
    _tc_a_body,
    out_shape=(jax.ShapeDtypeStruct((2, N), jnp.float32),
               jax.ShapeDtypeStruct((N, F), jnp.float32)))

_tc_c = pl.pallas_call(
    _tc_c_body,
    out_shape=(jax.ShapeDtypeStruct((N, 1), jnp.float32),
               jax.ShapeDtypeStruct((N, F), jnp.float32)))

_tc_e = pl.pallas_call(
    _tc_e_body,
    out_shape=(jax.ShapeDtypeStruct((N, F), jnp.float32),
               jax.ShapeDtypeStruct((N, F), jnp.float32),
               jax.ShapeDtypeStruct((N, F), jnp.float32)))

_tc_g = pl.pallas_call(
    _tc_g_body,
    out_shape=jax.ShapeDtypeStruct((N, F), jnp.float32))


def kernel(x, edge_index, edge_weight, W_attn, b_attn, W1, b1, W2, b2,
           g1, beta1, g2, beta2, Wf1, bf1, Wf2, bf2):
    src = edge_index[0].astype(jnp.int32)
    dst = edge_index[1].astype(jnp.int32)
    wcat = jnp.stack([W_attn[:F, 0], W_attn[F:, 0]])
    a12, h1pre = _tc_a(x, wcat, b_attn.reshape(1, 1), W1)
    a1 = a12[0]
    a2 = a12[1]
    w_e, degp = _edge_w_deg(src, dst, edge_weight, a1, a2)
    dinv_col, h1s = _tc_c(degp.reshape(2, N).T, h1pre)
    agg1 = _edge_agg(src, dst, w_e, h1s)
    h, h2pre, h2s = _tc_e(agg1, h1pre, dinv_col, b1.reshape(1, F),
                          g1.reshape(1, F), beta1.reshape(1, F), W2)
    agg2 = _edge_agg(src, dst, w_e, h2s)
    out = _tc_g(agg2, h2pre, dinv_col, b2.reshape(1, F), g2.reshape(1, F),
                beta2.reshape(1, F), x, h, Wf1, bf1.reshape(1, F),
                Wf2, bf2.reshape(1, F))
    return out


# speedup vs baseline: 21.0628x; 1.8693x over previous
import functools

import jax
import jax.numpy as jnp
from jax import lax
from jax.experimental import pallas as pl
from jax.experimental.pallas import tpu as pltpu
from jax.experimental.pallas import tpu_sc as plsc

N = 10000
E = 320000
F = 128
NW = 32
CH = E // NW
K = 80
WPW = CH // K

_mesh = plsc.VectorSubcoreMesh(core_axis_name="c", subcore_axis_name="s")


@functools.partial(
    pl.kernel, mesh=_mesh,
    out_type=(jax.ShapeDtypeStruct((E,), jnp.float32),
              jax.ShapeDtypeStruct((2 * N,), jnp.float32)),
    scratch_types=[
        pltpu.VMEM((CH,), jnp.int32),
        pltpu.VMEM((CH,), jnp.int32),
        pltpu.VMEM((CH,), jnp.float32),
        pltpu.VMEM((CH,), jnp.float32),
        pltpu.VMEM((K,), jnp.int32),
        pltpu.VMEM((K,), jnp.int32),
        pltpu.VMEM((K,), jnp.float32),
        pltpu.VMEM((K,), jnp.float32),
        pltpu.VMEM((K,), jnp.float32),
        pltpu.VMEM((K,), jnp.float32),
        pltpu.VMEM((640,), jnp.float32),
        pltpu.VMEM_SHARED((N,), jnp.float32),
        pltpu.SemaphoreType.DMA,
        pltpu.SemaphoreType.DMA,
        pltpu.SemaphoreType.DMA,
        pltpu.SemaphoreType.DMA,
    ],
)
def _edge_w_deg(src_hbm, dst_hbm, ew_hbm, a1_hbm, a2_hbm, w_out, deg_out,
                sidx_all, didx_all, ewv, wv_all, didx0, didx1,
                a1v0, a1v1, a2v0, a2v1,
                zbuf, deg_sp, semA0, semA1, semB0, semB1):
    c = lax.axis_index("c")
    s = lax.axis_index("s")
    wid = s * 2 + c
    a1b = (a1v0, a1v1)
    a2b = (a2v0, a2v1)
    didxb = (didx0, didx1)
    semA = (semA0, semA1)
    semB = (semB0, semB1)

    chunk = pl.multiple_of(wid * CH, 8)
    pltpu.sync_copy(src_hbm.at[pl.ds(chunk, CH)], sidx_all)
    pltpu.sync_copy(dst_hbm.at[pl.ds(chunk, CH)], didx_all)
    pltpu.sync_copy(ew_hbm.at[pl.ds(chunk, CH)], ewv)
    pltpu.async_copy(a1_hbm.at[sidx_all.at[pl.ds(0, K)]], a1v0, semA0)
    pltpu.async_copy(a2_hbm.at[didx_all.at[pl.ds(0, K)]], a2v0, semB0)

    zero = jnp.zeros((16,), jnp.float32)
    for q in range(40):
        zbuf[pl.ds(q * 16, 16)] = zero
    base = pl.multiple_of(s * 624, 8)
    pltpu.sync_copy(zbuf.at[pl.ds(0, 624)], deg_sp.at[pl.ds(base, 624)])

    @pl.when(s == 0)
    def _():
        pltpu.sync_copy(zbuf.at[pl.ds(0, 16)], deg_sp.at[pl.ds(9984, 16)])

    plsc.subcore_barrier()

    def slot(t, rb, prefetch):
        off = pl.multiple_of(t * K, 16)
        pltpu.make_async_copy(a1_hbm.at[sidx_all.at[pl.ds(0, K)]],
                              a1b[rb], semA[rb]).wait()
        pltpu.make_async_copy(a2_hbm.at[didx_all.at[pl.ds(0, K)]],
                              a2b[rb], semB[rb]).wait()
        if prefetch:
            off1 = pl.multiple_of((t + 1) * K, 16)
            pltpu.async_copy(a1_hbm.at[sidx_all.at[pl.ds(off1, K)]],
                             a1b[1 - rb], semA[1 - rb])
            pltpu.async_copy(a2_hbm.at[didx_all.at[pl.ds(off1, K)]],
                             a2b[1 - rb], semB[1 - rb])
        for g in range(K // 16):
            didxb[rb][pl.ds(g * 16, 16)] = didx_all[pl.ds(off + g * 16, 16)]
        for g in range(K // 16):
            sl = pl.ds(g * 16, 16)
            z = a1b[rb][sl] + a2b[rb][sl]
            wv_all[pl.ds(off + g * 16, 16)] = ewv[pl.ds(off + g * 16, 16)] \
                / (1.0 + jnp.exp(-z))
        pltpu.sync_copy(wv_all.at[pl.ds(off, K)], deg_sp.at[didxb[rb]],
                        add=True)

    def body(i, carry):
        t = i * 2
        slot(t, 0, True)
        slot(t + 1, 1, True)
        return carry

    lax.fori_loop(0, (WPW - 1) // 2, body, 0)
    slot(WPW - 1, 0, False)

    pltpu.sync_copy(wv_all, w_out.at[pl.ds(chunk, CH)])
    plsc.subcore_barrier()

    obase = pl.multiple_of(c * N + s * 624, 8)
    pltpu.sync_copy(deg_sp.at[pl.ds(base, 624)], zbuf.at[pl.ds(0, 624)])
    pltpu.sync_copy(zbuf.at[pl.ds(0, 624)], deg_out.at[pl.ds(obase, 624)])

    @pl.when(s == 0)
    def _():
        otail = pl.multiple_of(c * N + 9984, 8)
        pltpu.sync_copy(deg_sp.at[pl.ds(9984, 16)], zbuf.at[pl.ds(624, 16)])
        pltpu.sync_copy(zbuf.at[pl.ds(624, 16)], deg_out.at[pl.ds(otail, 16)])


@functools.partial(
    pl.kernel, mesh=_mesh,
    out_type=jax.ShapeDtypeStruct((2 * N, F), jnp.float32),
    scratch_types=[
        pltpu.VMEM((CH,), jnp.int32),
        pltpu.VMEM((CH,), jnp.int32),
        pltpu.VMEM((CH,), jnp.float32),
        pltpu.VMEM((K,), jnp.int32),
        pltpu.VMEM((K,), jnp.int32),
        pltpu.VMEM((K, F), jnp.float32),
        pltpu.VMEM((K, F), jnp.float32),
        pltpu.VMEM_SHARED((N, F), jnp.float32),
        pltpu.SemaphoreType.DMA,
        pltpu.SemaphoreType.DMA,
    ],
)
def _edge_agg(src_hbm, dst_hbm, w_hbm, h_hbm, agg_out,
              sidx_all, didx_all, wv_all, didx0, didx1,
              rows0, rows1, acc_sp, semG0, semG1):
    c = lax.axis_index("c")
    s = lax.axis_index("s")
    wid = s * 2 + c
    rowb = (rows0, rows1)
    didxb = (didx0, didx1)
    semG = (semG0, semG1)

    chunk = pl.multiple_of(wid * CH, 8)
    pltpu.sync_copy(src_hbm.at[pl.ds(chunk, CH)], sidx_all)
    pltpu.sync_copy(dst_hbm.at[pl.ds(chunk, CH)], didx_all)
    pltpu.sync_copy(w_hbm.at[pl.ds(chunk, CH)], wv_all)
    pltpu.async_copy(h_hbm.at[sidx_all.at[pl.ds(0, K)]], rows0, semG0)

    zero = jnp.zeros((16,), jnp.float32)

    def zbody(r, carry):
        for q in range(F // 16):
            rows1[r, pl.ds(q * 16, 16)] = zero
        return carry

    lax.fori_loop(0, K, zbody, 0)
    rstart = pl.multiple_of(s * 624, 8)
    for jj in range(8):
        size = 80 if jj < 7 else 64
        pltpu.sync_copy(rows1.at[pl.ds(0, size)],
                        acc_sp.at[pl.ds(rstart + jj * 80, size)])

    @pl.when(s == 15)
    def _():
        pltpu.sync_copy(rows1.at[pl.ds(0, 16)], acc_sp.at[pl.ds(9984, 16)])

    plsc.subcore_barrier()

    def slot(t, rb, prefetch):
        rows = rowb[rb]
        off = pl.multiple_of(t * K, 16)
        pltpu.make_async_copy(h_hbm.at[sidx_all.at[pl.ds(0, K)]],
                              rows, semG[rb]).wait()
        if prefetch:
            off1 = pl.multiple_of((t + 1) * K, 16)
            pltpu.async_copy(h_hbm.at[sidx_all.at[pl.ds(off1, K)]],
                             rowb[1 - rb], semG[1 - rb])
        for g in range(K // 16):
            didxb[rb][pl.ds(g * 16, 16)] = didx_all[pl.ds(off + g * 16, 16)]

        def mbody(g, mc):
            wg = wv_all[pl.ds(off + g * 16, 16)]
            for l in range(16):
                wspl = jnp.broadcast_to(wg[l], (16,))
                r = g * 16 + l
                for q in range(F // 16):
                    sl = pl.ds(q * 16, 16)
                    rows[r, sl] = rows[r, sl] * wspl
            return mc

        lax.fori_loop(0, K // 16, mbody, 0)
        pltpu.sync_copy(rows, acc_sp.at[didxb[rb]], add=True)

    def body(i, carry):
        t = i * 2
        slot(t, 0, True)
        slot(t + 1, 1, True)
        return carry

    lax.fori_loop(0, (WPW - 1) // 2, body, 0)
    slot(WPW - 1, 0, False)
    plsc.subcore_barrier()

    for jj in range(8):
        size = 80 if jj < 7 else 64
        ro = rstart + jj * 80
        oo = pl.multiple_of(c * N + ro, 8)
        pltpu.sync_copy(acc_sp.at[pl.ds(ro, size)], rows1.at[pl.ds(0, size)])
        pltpu.sync_copy(rows1.at[pl.ds(0, size)], agg_out.at[pl.ds(oo, size)])

    @pl.when(s == 15)
    def _():
        oo = pl.multiple_of(c * N + 9984, 8)
        pltpu.sync_copy(acc_sp.at[pl.ds(9984, 16)], rows1.at[pl.ds(0, 16)])
        pltpu.sync_copy(rows1.at[pl.ds(0, 16)], agg_out.at[pl.ds(oo, 16)])


def _tc_a_body(x_ref, wcat_ref, b_ref, w1_ref, a12_ref, h1pre_ref):
    x = x_ref[...]
    a12 = lax.dot_general(wcat_ref[...], x, (((1,), (1,)), ((), ())),
                          preferred_element_type=jnp.float32)
    a12_ref[...] = a12 + jnp.concatenate(
        [jnp.zeros((1, 1), jnp.float32), b_ref[...]], axis=0)
    h1pre_ref[...] = jnp.dot(x, w1_ref[...], preferred_element_type=jnp.float32)


def _tc_c_body(degp_ref, h1pre_ref, dinv_ref, h1s_ref):
    deg = degp_ref[:, 0:1] + degp_ref[:, 1:2] + 1.0
    dinv = lax.rsqrt(deg)
    dinv_ref[...] = dinv
    h1s_ref[...] = h1pre_ref[...] * dinv


def _bn_relu(conv, g_ref, beta_ref):
    r = jnp.maximum(conv, 0.0)
    mu = jnp.mean(r, axis=0, keepdims=True)
    var = jnp.mean(r * r, axis=0, keepdims=True) - mu * mu
    return g_ref[...] * (r - mu) * lax.rsqrt(var + 1e-5) + beta_ref[...]


def _tc_e_body(aggp_ref, h1pre_ref, dinv_ref, b1_ref, g1_ref, beta1_ref,
               w2_ref, h_ref, h2pre_ref, h2s_ref):
    dinv = dinv_ref[...]
    conv1 = dinv * (aggp_ref[0:N] + aggp_ref[N:2 * N]) \
        + (dinv * dinv) * h1pre_ref[...] + b1_ref[...]
    h = _bn_relu(conv1, g1_ref, beta1_ref)
    h_ref[...] = h
    h2pre = jnp.dot(h, w2_ref[...], preferred_element_type=jnp.float32)
    h2pre_ref[...] = h2pre
    h2s_ref[...] = h2pre * dinv


def _tc_g_body(aggp_ref, h2pre_ref, dinv_ref, b2_ref, g2_ref, beta2_ref,
               x_ref, h_ref, wf1_ref, bf1_ref, wf2_ref, bf2_ref, out_ref):
    dinv = dinv_ref[...]
    conv2 = dinv * (aggp_ref[0:N] + aggp_ref[N:2 * N]) \
        + (dinv * dinv) * h2pre_ref[...] + b2_ref[...]
    h2 = _bn_relu(conv2, g2_ref, beta2_ref)
    wf1 = wf1_ref[...]
    z = jnp.dot(x_ref[...], wf1[0:F], preferred_element_type=jnp.float32) \
        + jnp.dot(h_ref[...], wf1[F:2 * F], preferred_element_type=jnp.float32) \
        + jnp.dot(h2, wf1[2 * F:3 * F], preferred_element_type=jnp.float32) \
        + bf1_ref[...]
    z = jnp.maximum(z, 0.0)
    out = jnp.dot(z, wf2_ref[...], preferred_element_type=jnp.float32) + bf2_ref[...]
    out_ref[...] = jnp.maximum(out, 0.0)


_tc_a = pl.pallas_call(
    _tc_a_body,
    out_shape=(jax.ShapeDtypeStruct((2, N), jnp.float32),
               jax.ShapeDtypeStruct((N, F), jnp.float32)))

_tc_c = pl.pallas_call(
    _tc_c_body,
    out_shape=(jax.ShapeDtypeStruct((N, 1), jnp.float32),
               jax.ShapeDtypeStruct((N, F), jnp.float32)))

_tc_e = pl.pallas_call(
    _tc_e_body,
    out_shape=(jax.ShapeDtypeStruct((N, F), jnp.float32),
               jax.ShapeDtypeStruct((N, F), jnp.float32),
               jax.ShapeDtypeStruct((N, F), jnp.float32)))

_tc_g = pl.pallas_call(
    _tc_g_body,
    out_shape=jax.ShapeDtypeStruct((N, F), jnp.float32))


def kernel(x, edge_index, edge_weight, W_attn, b_attn, W1, b1, W2, b2,
           g1, beta1, g2, beta2, Wf1, bf1, Wf2, bf2):
    src3 = edge_index[0].astype(jnp.int32)
    dst3 = edge_index[1].astype(jnp.int32)
    wcat = jnp.stack([W_attn[:F, 0], W_attn[F:, 0]])
    a12, h1pre = _tc_a(x, wcat, b_attn.reshape(1, 1), W1)
    a1 = a12[0]
    a2 = a12[1]
    w_e, degp = _edge_w_deg(src3, dst3, edge_weight, a1, a2)
    dinv_col, h1s = _tc_c(degp.reshape(2, N).T, h1pre)
    agg1 = _edge_agg(src3, dst3, w_e, h1s)
    h, h2pre, h2s = _tc_e(agg1, h1pre, dinv_col, b1.reshape(1, F),
                          g1.reshape(1, F), beta1.reshape(1, F), W2)
    agg2 = _edge_agg(src3, dst3, w_e, h2s)
    out = _tc_g(agg2, h2pre, dinv_col, b2.reshape(1, F), g2.reshape(1, F),
                beta2.reshape(1, F), x, h, Wf1, bf1.reshape(1, F),
                Wf2, bf2.reshape(1, F))
    return out
